# Initial kernel scaffold; baseline (speedup 1.0000x reference)
import jax
import jax.numpy as jnp
from jax.experimental import pallas as pl


def kernel(feat, W1, attn_l1, attn_r1, bias1, W2, attn_l2, attn_r2, bias2, edge_index):
    raise NotImplementedError("write your pallas kernel here")


# speedup vs baseline: 16.5933x; 16.5933x over previous
import functools

import jax
import jax.numpy as jnp
from jax import lax
from jax.experimental import pallas as pl
from jax.experimental.pallas import tpu as pltpu
from jax.experimental.pallas import tpu_sc as plsc

N = 10000
E = 320000
D = 128
H1 = 8
O1 = 16
C = 40
CP = 48

NC = 2
NS = 16
L = 16

B1 = 128
ET1 = E // 4
NB1 = ET1 // B1
NQ = N // 4
ZC1 = 2504
ZR1 = 2440
ZB1R = 488
DQ = 2504
DR = 2488

B2 = 80
ES2A = E // NS
NB2A = ES2A // B2
ES2B = E // (NC * NS)
NB2B = ES2B // B2
ZC2 = 632
ZR2 = 520

f32 = jnp.float32
i32 = jnp.int32

_MESH = plsc.VectorSubcoreMesh(
    core_axis_name="c", subcore_axis_name="s", num_cores=NC, num_subcores=NS
)
_SC_PARAMS = pltpu.CompilerParams(
    needs_layout_passes=False, use_tc_tiling_on_sc=False
)


def _zero_ref(ref, nrows):
  zero = jnp.zeros((L,), f32)

  def body(i, _):
    ref[pl.ds(i * L, L)] = zero
    return 0

  lax.fori_loop(0, nrows, body, 0)


def _vec_loop(n, body):
  def wrap(i, _):
    body(i * L)
    return 0

  lax.fori_loop(0, n, wrap, 0)


def _t1_body(feat_ref, w1_ref, g1_ref, h_ref, eler_ref):
  h = jnp.dot(feat_ref[...], w1_ref[...], preferred_element_type=f32)
  h_ref[...] = h
  eler_ref[...] = jnp.dot(h, g1_ref[...], preferred_element_type=f32)


def _t1(feat, W1, G1):
  nb = 2000
  return pl.pallas_call(
      _t1_body,
      grid=(N // nb,),
      in_specs=[
          pl.BlockSpec((nb, D), lambda i: (i, 0)),
          pl.BlockSpec((D, D), lambda i: (0, 0)),
          pl.BlockSpec((D, 2 * H1), lambda i: (0, 0)),
      ],
      out_specs=[
          pl.BlockSpec((nb, D), lambda i: (i, 0)),
          pl.BlockSpec((nb, 2 * H1), lambda i: (i, 0)),
      ],
      out_shape=[
          jax.ShapeDtypeStruct((N, D), f32),
          jax.ShapeDtypeStruct((N, 2 * H1), f32),
      ],
  )(feat, W1, G1)


def _t2_body(agg_ref, b1_ref, w2_ref, g2_ref, h2_ref, eler2_ref):
  x = agg_ref[...] + b1_ref[...]
  x = jnp.where(x > 0, x, jnp.exp(x) - 1.0)
  h2 = jnp.dot(x, w2_ref[...], preferred_element_type=f32)
  h2_ref[...] = h2
  eler2_ref[...] = jnp.dot(h2, g2_ref[...], preferred_element_type=f32)


def _t2(agg1, bias1, W2p, G2):
  nb = 2000
  return pl.pallas_call(
      _t2_body,
      grid=(N // nb,),
      in_specs=[
          pl.BlockSpec((nb, D), lambda i: (i, 0)),
          pl.BlockSpec((1, D), lambda i: (0, 0)),
          pl.BlockSpec((D, CP), lambda i: (0, 0)),
          pl.BlockSpec((CP, 8), lambda i: (0, 0)),
      ],
      out_specs=[
          pl.BlockSpec((nb, CP), lambda i: (i, 0)),
          pl.BlockSpec((nb, 8), lambda i: (i, 0)),
      ],
      out_shape=[
          jax.ShapeDtypeStruct((N, CP), f32),
          jax.ShapeDtypeStruct((N, 8), f32),
      ],
  )(agg1, bias1, W2p, G2)


def _t3_body(p0_ref, p1_ref, b2_ref, out_ref):
  x = p0_ref[...] + p1_ref[...] + b2_ref[...]
  out_ref[...] = jnp.where(x > 0, x, jnp.exp(x) - 1.0)


def _t3(p0, p1, b2p):
  nb = 2000
  return pl.pallas_call(
      _t3_body,
      grid=(N // nb,),
      in_specs=[
          pl.BlockSpec((nb, CP), lambda i: (i, 0)),
          pl.BlockSpec((nb, CP), lambda i: (i, 0)),
          pl.BlockSpec((1, CP), lambda i: (0, 0)),
      ],
      out_specs=pl.BlockSpec((nb, CP), lambda i: (i, 0)),
      out_shape=jax.ShapeDtypeStruct((N, CP), f32),
  )(p0, p1, b2p)


def _s1_body(elT, erT, h1f, srcE, dstE, out_hbm,
             el_v, er_v, srs_v, tmp_v, srcb, dstb, idxo, alb, hrows, zb,
             comb, outsh, sem):
  c = lax.axis_index("c")
  s = lax.axis_index("s")
  hg = s // 4
  q = s % 4
  h = c * 4 + hg
  hN = pl.multiple_of(h * N, 8)

  pltpu.sync_copy(elT.at[pl.ds(hN, N)], el_v)
  pltpu.sync_copy(erT.at[pl.ds(hN, N)], er_v)
  _zero_ref(srs_v, N // L)

  def p1(j, _):
    base = pl.multiple_of(q * ET1 + j * B1, 8)
    pltpu.sync_copy(srcE.at[pl.ds(base, B1)], srcb)
    pltpu.sync_copy(dstE.at[pl.ds(base, B1)], dstb)

    def step(o):
      s16 = srcb[pl.ds(o, L)]
      d16 = dstb[pl.ds(o, L)]
      e = plsc.load_gather(el_v, [s16]) + plsc.load_gather(er_v, [d16])
      plsc.addupdate_scatter(srs_v, [d16], jnp.exp(e))

    _vec_loop(B1 // L, step)
    return 0

  lax.fori_loop(0, NB1, p1, 0)

  sN = pl.multiple_of(s * N, 8)
  pltpu.sync_copy(srs_v, comb.at[pl.ds(sN, N)])
  plsc.subcore_barrier()
  g0 = pl.multiple_of(4 * hg * N, 8)
  pltpu.sync_copy(comb.at[pl.ds(g0, N)], srs_v)
  for r in range(1, 4):
    gr = pl.multiple_of((4 * hg + r) * N, 8)
    pltpu.sync_copy(comb.at[pl.ds(gr, N)], tmp_v)

    def acc(o):
      srs_v[pl.ds(o, L)] = srs_v[pl.ds(o, L)] + tmp_v[pl.ds(o, L)]

    _vec_loop(N // L, acc)

  def rcp(o):
    srs_v[pl.ds(o, L)] = 1.0 / (srs_v[pl.ds(o, L)] + 1e-9)

  _vec_loop(N // L, rcp)

  zero = jnp.zeros((L,), f32)

  def zrow(i, _):
    zb[i] = zero
    return 0

  lax.fori_loop(0, ZB1R, zrow, 0)
  zoff = s * ZC1

  def zcp(i, _):
    pltpu.sync_copy(zb, outsh.at[pl.ds(zoff + i * ZB1R, ZB1R)])
    return 0

  lax.fori_loop(0, ZR1 // ZB1R, zcp, 0)

  @pl.when(s < NS - 1)
  def _():
    pltpu.sync_copy(zb.at[pl.ds(0, ZC1 - ZR1)],
                    outsh.at[pl.ds(zoff + ZR1, ZC1 - ZR1)])

  plsc.subcore_barrier()

  def p3(j, _):
    base = pl.multiple_of(q * ET1 + j * B1, 8)
    pltpu.sync_copy(srcE.at[pl.ds(base, B1)], srcb)
    pltpu.sync_copy(dstE.at[pl.ds(base, B1)], dstb)

    def mkidx(o):
      idxo[pl.ds(o, L)] = srcb[pl.ds(o, L)] + hN

    _vec_loop(B1 // L, mkidx)
    cp = pltpu.async_copy(h1f.at[idxo], hrows, sem)

    def alpha(o):
      s16 = srcb[pl.ds(o, L)]
      d16 = dstb[pl.ds(o, L)]
      e = plsc.load_gather(el_v, [s16]) + plsc.load_gather(er_v, [d16])
      alb[pl.ds(o, L)] = jnp.exp(e) * plsc.load_gather(srs_v, [d16])

    _vec_loop(B1 // L, alpha)
    cp.wait()

    def mul(o):
      for k in range(L):
        bc = plsc.load_gather(alb, [jnp.full((L,), o + k, dtype=i32)])
        hrows[o + k] = hrows[o + k] * bc

    _vec_loop(B1 // L, mul)

    def mkdst(o):
      idxo[pl.ds(o, L)] = dstb[pl.ds(o, L)] + hg * N

    _vec_loop(B1 // L, mkdst)
    pltpu.sync_copy(hrows, outsh.at[idxo], add=True)
    return 0

  lax.fori_loop(0, NB1, p3, 0)
  plsc.subcore_barrier()

  doff = q * DQ
  pltpu.sync_copy(outsh.at[pl.ds(hg * N + doff, DR)],
                  out_hbm.at[pl.ds(hN + doff, DR)])

  @pl.when(q < 3)
  def _():
    pltpu.sync_copy(outsh.at[pl.ds(hg * N + doff + DR, DQ - DR)],
                    out_hbm.at[pl.ds(hN + doff + DR, DQ - DR)])


def _s1(elT, erT, h1f, src, dst):
  return pl.kernel(
      _s1_body,
      out_type=jax.ShapeDtypeStruct((H1 * N, O1), f32),
      mesh=_MESH,
      compiler_params=_SC_PARAMS,
      scratch_types=[
          pltpu.VMEM((N,), f32),
          pltpu.VMEM((N,), f32),
          pltpu.VMEM((N,), f32),
          pltpu.VMEM((N,), f32),
          pltpu.VMEM((B1,), i32),
          pltpu.VMEM((B1,), i32),
          pltpu.VMEM((B1,), i32),
          pltpu.VMEM((B1,), f32),
          pltpu.VMEM((B1, O1), f32),
          pltpu.VMEM((ZB1R, O1), f32),
          pltpu.VMEM_SHARED((NS * N,), f32),
          pltpu.VMEM_SHARED((4 * N, O1), f32),
          pltpu.SemaphoreType.DMA,
      ],
  )(elT, erT, h1f, src, dst)


def _s2_body(el2, er2, h2f, srcE, dstE, out_hbm,
             el_v, er_v, srs_v, tmp_v, srcb, dstb, alb, hrows, zb,
             comb, outsh, sem):
  c = lax.axis_index("c")
  s = lax.axis_index("s")

  pltpu.sync_copy(el2, el_v)
  pltpu.sync_copy(er2, er_v)
  _zero_ref(srs_v, N // L)

  def p1(j, _):
    base = pl.multiple_of(s * ES2A + j * B2, 8)
    pltpu.sync_copy(srcE.at[pl.ds(base, B2)], srcb)
    pltpu.sync_copy(dstE.at[pl.ds(base, B2)], dstb)

    def step(o):
      s16 = srcb[pl.ds(o, L)]
      d16 = dstb[pl.ds(o, L)]
      e = plsc.load_gather(el_v, [s16]) + plsc.load_gather(er_v, [d16])
      plsc.addupdate_scatter(srs_v, [d16], jnp.exp(e))

    _vec_loop(B2 // L, step)
    return 0

  lax.fori_loop(0, NB2A, p1, 0)

  sN = pl.multiple_of(s * N, 8)
  pltpu.sync_copy(srs_v, comb.at[pl.ds(sN, N)])
  plsc.subcore_barrier()
  pltpu.sync_copy(comb.at[pl.ds(0, N)], srs_v)
  for r in range(1, NS):
    gr = pl.multiple_of(r * N, 8)
    pltpu.sync_copy(comb.at[pl.ds(gr, N)], tmp_v)

    def acc(o):
      srs_v[pl.ds(o, L)] = srs_v[pl.ds(o, L)] + tmp_v[pl.ds(o, L)]

    _vec_loop(N // L, acc)

  def rcp(o):
    srs_v[pl.ds(o, L)] = 1.0 / (srs_v[pl.ds(o, L)] + 1e-9)

  _vec_loop(N // L, rcp)

  zero = jnp.zeros((L,), f32)

  def zrow(i, _):
    for k in range(CP // L):
      zb[i, pl.ds(k * L, L)] = zero
    return 0

  lax.fori_loop(0, ZR2, zrow, 0)
  zoff = s * ZC2
  pltpu.sync_copy(zb, outsh.at[pl.ds(zoff, ZR2)])

  @pl.when(s < NS - 1)
  def _():
    pltpu.sync_copy(zb.at[pl.ds(0, ZC2 - ZR2)],
                    outsh.at[pl.ds(zoff + ZR2, ZC2 - ZR2)])

  plsc.subcore_barrier()

  gw = c * NS + s

  def p3(j, _):
    base = pl.multiple_of(gw * ES2B + j * B2, 8)
    pltpu.sync_copy(srcE.at[pl.ds(base, B2)], srcb)
    pltpu.sync_copy(dstE.at[pl.ds(base, B2)], dstb)
    cp = pltpu.async_copy(h2f.at[srcb], hrows, sem)

    def alpha(o):
      s16 = srcb[pl.ds(o, L)]
      d16 = dstb[pl.ds(o, L)]
      e = plsc.load_gather(el_v, [s16]) + plsc.load_gather(er_v, [d16])
      alb[pl.ds(o, L)] = jnp.exp(e) * plsc.load_gather(srs_v, [d16])

    _vec_loop(B2 // L, alpha)
    cp.wait()

    def mul(o):
      for k in range(L):
        bc = plsc.load_gather(alb, [jnp.full((L,), o + k, dtype=i32)])
        for kk in range(CP // L):
          sl = pl.ds(kk * L, L)
          hrows[o + k, sl] = hrows[o + k, sl] * bc

    _vec_loop(B2 // L, mul)
    pltpu.sync_copy(hrows, outsh.at[dstb], add=True)
    return 0

  lax.fori_loop(0, NB2B, p3, 0)
  plsc.subcore_barrier()

  pltpu.sync_copy(outsh.at[pl.ds(zoff, ZR2)],
                  out_hbm.at[pl.ds(c * N + zoff, ZR2)])

  @pl.when(s < NS - 1)
  def _():
    pltpu.sync_copy(outsh.at[pl.ds(zoff + ZR2, ZC2 - ZR2)],
                    out_hbm.at[pl.ds(c * N + zoff + ZR2, ZC2 - ZR2)])


def _s2(el2, er2, h2f, src, dst):
  return pl.kernel(
      _s2_body,
      out_type=jax.ShapeDtypeStruct((NC * N, CP), f32),
      mesh=_MESH,
      compiler_params=_SC_PARAMS,
      scratch_types=[
          pltpu.VMEM((N,), f32),
          pltpu.VMEM((N,), f32),
          pltpu.VMEM((N,), f32),
          pltpu.VMEM((N,), f32),
          pltpu.VMEM((B2,), i32),
          pltpu.VMEM((B2,), i32),
          pltpu.VMEM((B2,), f32),
          pltpu.VMEM((B2, CP), f32),
          pltpu.VMEM((ZR2, CP), f32),
          pltpu.VMEM_SHARED((NS * N,), f32),
          pltpu.VMEM_SHARED((N, CP), f32),
          pltpu.SemaphoreType.DMA,
      ],
  )(el2, er2, h2f, src, dst)


def kernel(feat, W1, attn_l1, attn_r1, bias1, W2, attn_l2, attn_r2, bias2,
           edge_index):
  src = edge_index[0]
  dst = edge_index[1]

  eye8 = jnp.eye(H1, dtype=f32)
  Gl1 = (eye8[:, None, :] * attn_l1[:, :, None]).reshape(D, H1)
  Gr1 = (eye8[:, None, :] * attn_r1[:, :, None]).reshape(D, H1)
  G1 = jnp.concatenate([Gl1, Gr1], axis=1)

  h1, eler1 = _t1(feat, W1, G1)
  elT1 = eler1[:, :H1].T.reshape(-1)
  erT1 = eler1[:, H1:].T.reshape(-1)
  h1f = h1.reshape(N, H1, O1).transpose(1, 0, 2).reshape(H1 * N, O1)

  agg1f = _s1(elT1, erT1, h1f, src, dst)
  agg1 = agg1f.reshape(H1, N, O1).transpose(1, 0, 2).reshape(N, D)

  W2p = jnp.pad(W2, ((0, 0), (0, CP - C)))
  G2 = jnp.zeros((CP, 8), f32)
  G2 = G2.at[:C, 0].set(attn_l2[0]).at[:C, 1].set(attn_r2[0])
  h2p, eler2 = _t2(agg1, bias1.reshape(1, D), W2p, G2)

  out2 = _s2(eler2[:, 0], eler2[:, 1], h2p, src, dst)

  b2p = jnp.pad(bias2, (0, CP - C)).reshape(1, CP)
  out48 = _t3(out2[:N], out2[N:], b2p)
  return out48[:, :C]


# speedup vs baseline: 29.3154x; 1.7667x over previous
import functools

import jax
import jax.numpy as jnp
from jax import lax
from jax.experimental import pallas as pl
from jax.experimental.pallas import tpu as pltpu
from jax.experimental.pallas import tpu_sc as plsc

N = 10000
E = 320000
D = 128
H1 = 8
O1 = 16
C = 40
CP = 48

NC = 2
NS = 16
L = 16

B1 = 128
ET1 = E // 4
NB1 = ET1 // B1
IB1 = 3200
NBT1 = ET1 // IB1
BPB1 = IB1 // B1
NQ = N // 4
ZC1 = 2504
ZR1 = 2440
ZB1R = 488
DQ = 2504
DR = 2488

B2 = 80
ES2A = E // NS
NB2A = ES2A // B2
ES2B = E // (NC * NS)
NB2B = ES2B // B2
ZC2 = 632
ZR2 = 520

f32 = jnp.float32
i32 = jnp.int32

_MESH = plsc.VectorSubcoreMesh(
    core_axis_name="c", subcore_axis_name="s", num_cores=NC, num_subcores=NS
)
_SC_PARAMS = pltpu.CompilerParams(
    needs_layout_passes=False, use_tc_tiling_on_sc=False
)


def _zero_ref(ref, nrows):
  zero = jnp.zeros((L,), f32)

  def body(i, _):
    ref[pl.ds(i * L, L)] = zero
    return 0

  lax.fori_loop(0, nrows, body, 0)


def _vec_loop(n, body):
  def wrap(i, _):
    body(i * L)
    return 0

  lax.fori_loop(0, n, wrap, 0)


def _t1_body(feat_ref, w1_ref, g1_ref, h_ref, eler_ref):
  h = jnp.dot(feat_ref[...], w1_ref[...], preferred_element_type=f32)
  h_ref[...] = h
  eler_ref[...] = jnp.dot(h, g1_ref[...], preferred_element_type=f32)


def _t1(feat, W1, G1):
  nb = 2000
  return pl.pallas_call(
      _t1_body,
      grid=(N // nb,),
      in_specs=[
          pl.BlockSpec((nb, D), lambda i: (i, 0)),
          pl.BlockSpec((D, D), lambda i: (0, 0)),
          pl.BlockSpec((D, 2 * H1), lambda i: (0, 0)),
      ],
      out_specs=[
          pl.BlockSpec((nb, D), lambda i: (i, 0)),
          pl.BlockSpec((nb, 2 * H1), lambda i: (i, 0)),
      ],
      out_shape=[
          jax.ShapeDtypeStruct((N, D), f32),
          jax.ShapeDtypeStruct((N, 2 * H1), f32),
      ],
  )(feat, W1, G1)


def _t2_body(agg_ref, b1_ref, w2_ref, g2_ref, h2_ref, eler2_ref):
  x = agg_ref[...] + b1_ref[...]
  x = jnp.where(x > 0, x, jnp.exp(x) - 1.0)
  h2 = jnp.dot(x, w2_ref[...], preferred_element_type=f32)
  h2_ref[...] = h2
  eler2_ref[...] = jnp.dot(h2, g2_ref[...], preferred_element_type=f32)


def _t2(agg1, bias1, W2p, G2):
  nb = 2000
  return pl.pallas_call(
      _t2_body,
      grid=(N // nb,),
      in_specs=[
          pl.BlockSpec((nb, D), lambda i: (i, 0)),
          pl.BlockSpec((1, D), lambda i: (0, 0)),
          pl.BlockSpec((D, CP), lambda i: (0, 0)),
          pl.BlockSpec((CP, 8), lambda i: (0, 0)),
      ],
      out_specs=[
          pl.BlockSpec((nb, CP), lambda i: (i, 0)),
          pl.BlockSpec((nb, 8), lambda i: (i, 0)),
      ],
      out_shape=[
          jax.ShapeDtypeStruct((N, CP), f32),
          jax.ShapeDtypeStruct((N, 8), f32),
      ],
  )(agg1, bias1, W2p, G2)


def _t3_body(p0_ref, p1_ref, b2_ref, out_ref):
  x = p0_ref[...] + p1_ref[...] + b2_ref[...]
  out_ref[...] = jnp.where(x > 0, x, jnp.exp(x) - 1.0)


def _t3(p0, p1, b2p):
  nb = 2000
  return pl.pallas_call(
      _t3_body,
      grid=(N // nb,),
      in_specs=[
          pl.BlockSpec((nb, CP), lambda i: (i, 0)),
          pl.BlockSpec((nb, CP), lambda i: (i, 0)),
          pl.BlockSpec((1, CP), lambda i: (0, 0)),
      ],
      out_specs=pl.BlockSpec((nb, CP), lambda i: (i, 0)),
      out_shape=jax.ShapeDtypeStruct((N, CP), f32),
  )(p0, p1, b2p)


def _s1_body(elT, erT, h1f, srcE, dstE, out_hbm,
             el_v, er_v, srs_v, tmp_v, srcbat, dstbat,
             iga, igb, isa, isb, alb, hra, hrb, zb,
             comb, outsh, semga, semgb, semsa, semsb):
  c = lax.axis_index("c")
  s = lax.axis_index("s")
  hg = s // 4
  q = s % 4
  h = c * 4 + hg
  hN = pl.multiple_of(h * N, 8)

  pltpu.sync_copy(elT.at[pl.ds(hN, N)], el_v)
  pltpu.sync_copy(erT.at[pl.ds(hN, N)], er_v)
  _zero_ref(srs_v, N // L)

  def p1(b, _):
    base = pl.multiple_of(q * ET1 + b * IB1, 8)
    pltpu.sync_copy(srcE.at[pl.ds(base, IB1)], srcbat)
    pltpu.sync_copy(dstE.at[pl.ds(base, IB1)], dstbat)

    def step(o):
      s16 = srcbat[pl.ds(o, L)]
      d16 = dstbat[pl.ds(o, L)]
      e = plsc.load_gather(el_v, [s16]) + plsc.load_gather(er_v, [d16])
      plsc.addupdate_scatter(srs_v, [d16], jnp.exp(e))

    _vec_loop(IB1 // L, step)
    return 0

  lax.fori_loop(0, NBT1, p1, 0)

  sN = pl.multiple_of(s * N, 8)
  pltpu.sync_copy(srs_v, comb.at[pl.ds(sN, N)])
  plsc.subcore_barrier()
  g0 = pl.multiple_of(4 * hg * N, 8)
  pltpu.sync_copy(comb.at[pl.ds(g0, N)], srs_v)
  for r in range(1, 4):
    gr = pl.multiple_of((4 * hg + r) * N, 8)
    pltpu.sync_copy(comb.at[pl.ds(gr, N)], tmp_v)

    def acc(o):
      srs_v[pl.ds(o, L)] = srs_v[pl.ds(o, L)] + tmp_v[pl.ds(o, L)]

    _vec_loop(N // L, acc)

  def rcp(o):
    srs_v[pl.ds(o, L)] = 1.0 / (srs_v[pl.ds(o, L)] + 1e-9)

  _vec_loop(N // L, rcp)

  zero = jnp.zeros((L,), f32)

  def zrow(i, _):
    zb[i] = zero
    return 0

  lax.fori_loop(0, ZB1R, zrow, 0)
  zoff = s * ZC1

  def zcp(i, _):
    pltpu.sync_copy(zb, outsh.at[pl.ds(zoff + i * ZB1R, ZB1R)])
    return 0

  lax.fori_loop(0, ZR1 // ZB1R, zcp, 0)

  @pl.when(s < NS - 1)
  def _():
    pltpu.sync_copy(zb.at[pl.ds(0, ZC1 - ZR1)],
                    outsh.at[pl.ds(zoff + ZR1, ZC1 - ZR1)])

  plsc.subcore_barrier()

  def issue_gather(j, ig, hr, sg):
    def mk(o):
      ig[pl.ds(o, L)] = srcbat[pl.ds(j * B1 + o, L)] + hN

    _vec_loop(B1 // L, mk)
    pltpu.async_copy(h1f.at[ig], hr, sg)

  def do_block(j, ig, hr, sg, isd, ss):
    def alpha(o):
      s16 = srcbat[pl.ds(j * B1 + o, L)]
      d16 = dstbat[pl.ds(j * B1 + o, L)]
      e = plsc.load_gather(el_v, [s16]) + plsc.load_gather(er_v, [d16])
      alb[pl.ds(o, L)] = jnp.exp(e) * plsc.load_gather(srs_v, [d16])

    _vec_loop(B1 // L, alpha)
    pltpu.make_async_copy(h1f.at[ig], hr, sg).wait()

    def mul(o):
      for k in range(L):
        bc = plsc.load_gather(alb, [jnp.full((L,), o + k, dtype=i32)])
        hr[o + k] = hr[o + k] * bc

    _vec_loop(B1 // L, mul)

    def mkdst(o):
      isd[pl.ds(o, L)] = dstbat[pl.ds(j * B1 + o, L)] + hg * N

    _vec_loop(B1 // L, mkdst)
    pltpu.async_copy(hr, outsh.at[isd], ss, add=True)

  def p3(b, _):
    base = pl.multiple_of(q * ET1 + b * IB1, 8)
    pltpu.sync_copy(srcE.at[pl.ds(base, IB1)], srcbat)
    pltpu.sync_copy(dstE.at[pl.ds(base, IB1)], dstbat)
    issue_gather(0, iga, hra, semga)

    def pair(i, _):
      j0 = 2 * i

      @pl.when(i > 0)
      def _():
        pltpu.make_async_copy(hrb, outsh.at[isb], semsb).wait()

      issue_gather(j0 + 1, igb, hrb, semgb)
      do_block(j0, iga, hra, semga, isa, semsa)
      do_block(j0 + 1, igb, hrb, semgb, isb, semsb)
      pltpu.make_async_copy(hra, outsh.at[isa], semsa).wait()
      issue_gather(j0 + 2, iga, hra, semga)
      return 0

    lax.fori_loop(0, BPB1 // 2, pair, 0)
    pltpu.make_async_copy(hrb, outsh.at[isb], semsb).wait()
    do_block(BPB1 - 1, iga, hra, semga, isa, semsa)
    pltpu.make_async_copy(hra, outsh.at[isa], semsa).wait()
    return 0

  lax.fori_loop(0, NBT1, p3, 0)
  plsc.subcore_barrier()

  doff = q * DQ
  pltpu.sync_copy(outsh.at[pl.ds(hg * N + doff, DR)],
                  out_hbm.at[pl.ds(hN + doff, DR)])

  @pl.when(q < 3)
  def _():
    pltpu.sync_copy(outsh.at[pl.ds(hg * N + doff + DR, DQ - DR)],
                    out_hbm.at[pl.ds(hN + doff + DR, DQ - DR)])


def _s1(elT, erT, h1f, src, dst):
  return pl.kernel(
      _s1_body,
      out_type=jax.ShapeDtypeStruct((H1 * N, O1), f32),
      mesh=_MESH,
      compiler_params=_SC_PARAMS,
      scratch_types=[
          pltpu.VMEM((N,), f32),
          pltpu.VMEM((N,), f32),
          pltpu.VMEM((N,), f32),
          pltpu.VMEM((N,), f32),
          pltpu.VMEM((IB1,), i32),
          pltpu.VMEM((IB1,), i32),
          pltpu.VMEM((B1,), i32),
          pltpu.VMEM((B1,), i32),
          pltpu.VMEM((B1,), i32),
          pltpu.VMEM((B1,), i32),
          pltpu.VMEM((B1,), f32),
          pltpu.VMEM((B1, O1), f32),
          pltpu.VMEM((B1, O1), f32),
          pltpu.VMEM((ZB1R, O1), f32),
          pltpu.VMEM_SHARED((NS * N,), f32),
          pltpu.VMEM_SHARED((4 * N, O1), f32),
          pltpu.SemaphoreType.DMA,
          pltpu.SemaphoreType.DMA,
          pltpu.SemaphoreType.DMA,
          pltpu.SemaphoreType.DMA,
      ],
  )(elT, erT, h1f, src, dst)


def _s2_body(el2, er2, h2f, srcE, dstE, out_hbm,
             el_v, er_v, srs_v, tmp_v, srcb, dstb, alb, hrows, zb,
             comb, outsh, sem):
  c = lax.axis_index("c")
  s = lax.axis_index("s")

  pltpu.sync_copy(el2, el_v)
  pltpu.sync_copy(er2, er_v)
  _zero_ref(srs_v, N // L)

  def p1(j, _):
    base = pl.multiple_of(s * ES2A + j * B2, 8)
    pltpu.sync_copy(srcE.at[pl.ds(base, B2)], srcb)
    pltpu.sync_copy(dstE.at[pl.ds(base, B2)], dstb)

    def step(o):
      s16 = srcb[pl.ds(o, L)]
      d16 = dstb[pl.ds(o, L)]
      e = plsc.load_gather(el_v, [s16]) + plsc.load_gather(er_v, [d16])
      plsc.addupdate_scatter(srs_v, [d16], jnp.exp(e))

    _vec_loop(B2 // L, step)
    return 0

  lax.fori_loop(0, NB2A, p1, 0)

  sN = pl.multiple_of(s * N, 8)
  pltpu.sync_copy(srs_v, comb.at[pl.ds(sN, N)])
  plsc.subcore_barrier()
  pltpu.sync_copy(comb.at[pl.ds(0, N)], srs_v)
  for r in range(1, NS):
    gr = pl.multiple_of(r * N, 8)
    pltpu.sync_copy(comb.at[pl.ds(gr, N)], tmp_v)

    def acc(o):
      srs_v[pl.ds(o, L)] = srs_v[pl.ds(o, L)] + tmp_v[pl.ds(o, L)]

    _vec_loop(N // L, acc)

  def rcp(o):
    srs_v[pl.ds(o, L)] = 1.0 / (srs_v[pl.ds(o, L)] + 1e-9)

  _vec_loop(N // L, rcp)

  zero = jnp.zeros((L,), f32)

  def zrow(i, _):
    for k in range(CP // L):
      zb[i, pl.ds(k * L, L)] = zero
    return 0

  lax.fori_loop(0, ZR2, zrow, 0)
  zoff = s * ZC2
  pltpu.sync_copy(zb, outsh.at[pl.ds(zoff, ZR2)])

  @pl.when(s < NS - 1)
  def _():
    pltpu.sync_copy(zb.at[pl.ds(0, ZC2 - ZR2)],
                    outsh.at[pl.ds(zoff + ZR2, ZC2 - ZR2)])

  plsc.subcore_barrier()

  gw = c * NS + s

  def p3(j, _):
    base = pl.multiple_of(gw * ES2B + j * B2, 8)
    pltpu.sync_copy(srcE.at[pl.ds(base, B2)], srcb)
    pltpu.sync_copy(dstE.at[pl.ds(base, B2)], dstb)
    cp = pltpu.async_copy(h2f.at[srcb], hrows, sem)

    def alpha(o):
      s16 = srcb[pl.ds(o, L)]
      d16 = dstb[pl.ds(o, L)]
      e = plsc.load_gather(el_v, [s16]) + plsc.load_gather(er_v, [d16])
      alb[pl.ds(o, L)] = jnp.exp(e) * plsc.load_gather(srs_v, [d16])

    _vec_loop(B2 // L, alpha)
    cp.wait()

    def mul(o):
      for k in range(L):
        bc = plsc.load_gather(alb, [jnp.full((L,), o + k, dtype=i32)])
        for kk in range(CP // L):
          sl = pl.ds(kk * L, L)
          hrows[o + k, sl] = hrows[o + k, sl] * bc

    _vec_loop(B2 // L, mul)
    pltpu.sync_copy(hrows, outsh.at[dstb], add=True)
    return 0

  lax.fori_loop(0, NB2B, p3, 0)
  plsc.subcore_barrier()

  pltpu.sync_copy(outsh.at[pl.ds(zoff, ZR2)],
                  out_hbm.at[pl.ds(c * N + zoff, ZR2)])

  @pl.when(s < NS - 1)
  def _():
    pltpu.sync_copy(outsh.at[pl.ds(zoff + ZR2, ZC2 - ZR2)],
                    out_hbm.at[pl.ds(c * N + zoff + ZR2, ZC2 - ZR2)])


def _s2(el2, er2, h2f, src, dst):
  return pl.kernel(
      _s2_body,
      out_type=jax.ShapeDtypeStruct((NC * N, CP), f32),
      mesh=_MESH,
      compiler_params=_SC_PARAMS,
      scratch_types=[
          pltpu.VMEM((N,), f32),
          pltpu.VMEM((N,), f32),
          pltpu.VMEM((N,), f32),
          pltpu.VMEM((N,), f32),
          pltpu.VMEM((B2,), i32),
          pltpu.VMEM((B2,), i32),
          pltpu.VMEM((B2,), f32),
          pltpu.VMEM((B2, CP), f32),
          pltpu.VMEM((ZR2, CP), f32),
          pltpu.VMEM_SHARED((NS * N,), f32),
          pltpu.VMEM_SHARED((N, CP), f32),
          pltpu.SemaphoreType.DMA,
      ],
  )(el2, er2, h2f, src, dst)


def kernel(feat, W1, attn_l1, attn_r1, bias1, W2, attn_l2, attn_r2, bias2,
           edge_index):
  src = edge_index[0]
  dst = edge_index[1]

  eye8 = jnp.eye(H1, dtype=f32)
  Gl1 = (eye8[:, None, :] * attn_l1[:, :, None]).reshape(D, H1)
  Gr1 = (eye8[:, None, :] * attn_r1[:, :, None]).reshape(D, H1)
  G1 = jnp.concatenate([Gl1, Gr1], axis=1)

  h1, eler1 = _t1(feat, W1, G1)
  elT1 = eler1[:, :H1].T.reshape(-1)
  erT1 = eler1[:, H1:].T.reshape(-1)
  h1f = h1.reshape(N, H1, O1).transpose(1, 0, 2).reshape(H1 * N, O1)

  agg1f = _s1(elT1, erT1, h1f, src, dst)
  agg1 = agg1f.reshape(H1, N, O1).transpose(1, 0, 2).reshape(N, D)

  W2p = jnp.pad(W2, ((0, 0), (0, CP - C)))
  G2 = jnp.zeros((CP, 8), f32)
  G2 = G2.at[:C, 0].set(attn_l2[0]).at[:C, 1].set(attn_r2[0])
  h2p, eler2 = _t2(agg1, bias1.reshape(1, D), W2p, G2)

  out2 = _s2(eler2[:, 0], eler2[:, 1], h2p, src, dst)

  b2p = jnp.pad(bias2, (0, CP - C)).reshape(1, CP)
  out48 = _t3(out2[:N], out2[N:], b2p)
  return out48[:, :C]


# speedup vs baseline: 36.9477x; 1.2604x over previous
import functools

import jax
import jax.numpy as jnp
from jax import lax
from jax.experimental import pallas as pl
from jax.experimental.pallas import tpu as pltpu
from jax.experimental.pallas import tpu_sc as plsc

N = 10000
E = 320000
D = 128
H1 = 8
O1 = 16
C = 40
CP = 48

NC = 2
NS = 16
L = 16

B1 = 128
ET1 = E // 4
NB1 = ET1 // B1
IB1 = 3200
NBT1 = ET1 // IB1
BPB1 = IB1 // B1
NQ = N // 4
ZC1 = 2504
ZR1 = 2440
ZB1R = 488
DQ = 2504
DR = 2488

B2 = 80
ES2A = E // NS
ES2B = E // (NC * NS)
IB2 = 2000
NBT2A = ES2A // IB2
NBT2B = ES2B // IB2
BPB2 = IB2 // B2
ZC2 = 632
ZR2 = 520

f32 = jnp.float32
i32 = jnp.int32

_MESH = plsc.VectorSubcoreMesh(
    core_axis_name="c", subcore_axis_name="s", num_cores=NC, num_subcores=NS
)
_SC_PARAMS = pltpu.CompilerParams(
    needs_layout_passes=False, use_tc_tiling_on_sc=False
)


def _zero_ref(ref, nrows):
  zero = jnp.zeros((L,), f32)

  def body(i, _):
    ref[pl.ds(i * L, L)] = zero
    return 0

  lax.fori_loop(0, nrows, body, 0)


def _vec_loop(n, body):
  def wrap(i, _):
    body(i * L)
    return 0

  lax.fori_loop(0, n, wrap, 0)


def _t1_body(feat_ref, w1_ref, g1_ref, h_ref, eler_ref):
  h = jnp.dot(feat_ref[...], w1_ref[...], preferred_element_type=f32)
  h_ref[...] = h
  eler_ref[...] = jnp.dot(h, g1_ref[...], preferred_element_type=f32)


def _t1(feat, W1, G1):
  nb = 2000
  return pl.pallas_call(
      _t1_body,
      grid=(N // nb,),
      in_specs=[
          pl.BlockSpec((nb, D), lambda i: (i, 0)),
          pl.BlockSpec((D, D), lambda i: (0, 0)),
          pl.BlockSpec((D, 2 * H1), lambda i: (0, 0)),
      ],
      out_specs=[
          pl.BlockSpec((nb, D), lambda i: (i, 0)),
          pl.BlockSpec((nb, 2 * H1), lambda i: (i, 0)),
      ],
      out_shape=[
          jax.ShapeDtypeStruct((N, D), f32),
          jax.ShapeDtypeStruct((N, 2 * H1), f32),
      ],
  )(feat, W1, G1)


def _t2_body(agg_ref, b1_ref, w2_ref, g2_ref, h2_ref, eler2_ref):
  x = agg_ref[...] + b1_ref[...]
  x = jnp.where(x > 0, x, jnp.exp(x) - 1.0)
  h2 = jnp.dot(x, w2_ref[...], preferred_element_type=f32)
  h2_ref[...] = h2
  eler2_ref[...] = jnp.dot(h2, g2_ref[...], preferred_element_type=f32)


def _t2(agg1, bias1, W2p, G2):
  nb = 2000
  return pl.pallas_call(
      _t2_body,
      grid=(N // nb,),
      in_specs=[
          pl.BlockSpec((nb, D), lambda i: (i, 0)),
          pl.BlockSpec((1, D), lambda i: (0, 0)),
          pl.BlockSpec((D, CP), lambda i: (0, 0)),
          pl.BlockSpec((CP, 8), lambda i: (0, 0)),
      ],
      out_specs=[
          pl.BlockSpec((nb, CP), lambda i: (i, 0)),
          pl.BlockSpec((nb, 8), lambda i: (i, 0)),
      ],
      out_shape=[
          jax.ShapeDtypeStruct((N, CP), f32),
          jax.ShapeDtypeStruct((N, 8), f32),
      ],
  )(agg1, bias1, W2p, G2)


def _t3_body(p0_ref, p1_ref, b2_ref, out_ref):
  x = p0_ref[...] + p1_ref[...] + b2_ref[...]
  out_ref[...] = jnp.where(x > 0, x, jnp.exp(x) - 1.0)


def _t3(p0, p1, b2p):
  nb = 2000
  return pl.pallas_call(
      _t3_body,
      grid=(N // nb,),
      in_specs=[
          pl.BlockSpec((nb, CP), lambda i: (i, 0)),
          pl.BlockSpec((nb, CP), lambda i: (i, 0)),
          pl.BlockSpec((1, CP), lambda i: (0, 0)),
      ],
      out_specs=pl.BlockSpec((nb, CP), lambda i: (i, 0)),
      out_shape=jax.ShapeDtypeStruct((N, CP), f32),
  )(p0, p1, b2p)


def _s1_body(elT, erT, h1f, srcE, dstE, out_hbm,
             el_v, er_v, srs_v, tmp_v, srcbat, dstbat,
             iga, igb, isa, isb, alb, hra, hrb, zb,
             comb, outsh, semga, semgb, semsa, semsb):
  c = lax.axis_index("c")
  s = lax.axis_index("s")
  hg = s // 4
  q = s % 4
  h = c * 4 + hg
  hN = pl.multiple_of(h * N, 8)

  pltpu.sync_copy(elT.at[pl.ds(hN, N)], el_v)
  pltpu.sync_copy(erT.at[pl.ds(hN, N)], er_v)
  _zero_ref(srs_v, N // L)

  def p1(b, _):
    base = pl.multiple_of(q * ET1 + b * IB1, 8)
    pltpu.sync_copy(srcE.at[pl.ds(base, IB1)], srcbat)
    pltpu.sync_copy(dstE.at[pl.ds(base, IB1)], dstbat)

    def step(o):
      s16 = srcbat[pl.ds(o, L)]
      d16 = dstbat[pl.ds(o, L)]
      e = plsc.load_gather(el_v, [s16]) + plsc.load_gather(er_v, [d16])
      plsc.addupdate_scatter(srs_v, [d16], jnp.exp(e))

    _vec_loop(IB1 // L, step)
    return 0

  lax.fori_loop(0, NBT1, p1, 0)

  sN = pl.multiple_of(s * N, 8)
  pltpu.sync_copy(srs_v, comb.at[pl.ds(sN, N)])
  plsc.subcore_barrier()
  g0 = pl.multiple_of(4 * hg * N, 8)
  pltpu.sync_copy(comb.at[pl.ds(g0, N)], srs_v)
  for r in range(1, 4):
    gr = pl.multiple_of((4 * hg + r) * N, 8)
    pltpu.sync_copy(comb.at[pl.ds(gr, N)], tmp_v)

    def acc(o):
      srs_v[pl.ds(o, L)] = srs_v[pl.ds(o, L)] + tmp_v[pl.ds(o, L)]

    _vec_loop(N // L, acc)

  def rcp(o):
    srs_v[pl.ds(o, L)] = 1.0 / (srs_v[pl.ds(o, L)] + 1e-9)

  _vec_loop(N // L, rcp)

  zero = jnp.zeros((L,), f32)

  def zrow(i, _):
    zb[i] = zero
    return 0

  lax.fori_loop(0, ZB1R, zrow, 0)
  zoff = s * ZC1

  def zcp(i, _):
    pltpu.sync_copy(zb, outsh.at[pl.ds(zoff + i * ZB1R, ZB1R)])
    return 0

  lax.fori_loop(0, ZR1 // ZB1R, zcp, 0)

  @pl.when(s < NS - 1)
  def _():
    pltpu.sync_copy(zb.at[pl.ds(0, ZC1 - ZR1)],
                    outsh.at[pl.ds(zoff + ZR1, ZC1 - ZR1)])

  plsc.subcore_barrier()

  def issue_gather(j, ig, hr, sg):
    def mk(o):
      ig[pl.ds(o, L)] = srcbat[pl.ds(j * B1 + o, L)] + hN

    _vec_loop(B1 // L, mk)
    pltpu.async_copy(h1f.at[ig], hr, sg)

  def do_block(j, ig, hr, sg, isd, ss):
    def alpha(o):
      s16 = srcbat[pl.ds(j * B1 + o, L)]
      d16 = dstbat[pl.ds(j * B1 + o, L)]
      e = plsc.load_gather(el_v, [s16]) + plsc.load_gather(er_v, [d16])
      alb[pl.ds(o, L)] = jnp.exp(e) * plsc.load_gather(srs_v, [d16])

    _vec_loop(B1 // L, alpha)
    pltpu.make_async_copy(h1f.at[ig], hr, sg).wait()

    def mul(o):
      for k in range(L):
        bc = plsc.load_gather(alb, [jnp.full((L,), o + k, dtype=i32)])
        hr[o + k] = hr[o + k] * bc

    _vec_loop(B1 // L, mul)

    def mkdst(o):
      isd[pl.ds(o, L)] = dstbat[pl.ds(j * B1 + o, L)] + hg * N

    _vec_loop(B1 // L, mkdst)
    pltpu.async_copy(hr, outsh.at[isd], ss, add=True)

  def p3(b, _):
    base = pl.multiple_of(q * ET1 + b * IB1, 8)
    pltpu.sync_copy(srcE.at[pl.ds(base, IB1)], srcbat)
    pltpu.sync_copy(dstE.at[pl.ds(base, IB1)], dstbat)
    issue_gather(0, iga, hra, semga)

    def pair(i, _):
      j0 = 2 * i

      @pl.when(i > 0)
      def _():
        pltpu.make_async_copy(hrb, outsh.at[isb], semsb).wait()

      issue_gather(j0 + 1, igb, hrb, semgb)
      do_block(j0, iga, hra, semga, isa, semsa)
      do_block(j0 + 1, igb, hrb, semgb, isb, semsb)
      pltpu.make_async_copy(hra, outsh.at[isa], semsa).wait()
      issue_gather(j0 + 2, iga, hra, semga)
      return 0

    lax.fori_loop(0, BPB1 // 2, pair, 0)
    pltpu.make_async_copy(hrb, outsh.at[isb], semsb).wait()
    do_block(BPB1 - 1, iga, hra, semga, isa, semsa)
    pltpu.make_async_copy(hra, outsh.at[isa], semsa).wait()
    return 0

  lax.fori_loop(0, NBT1, p3, 0)
  plsc.subcore_barrier()

  doff = q * DQ
  pltpu.sync_copy(outsh.at[pl.ds(hg * N + doff, DR)],
                  out_hbm.at[pl.ds(hN + doff, DR)])

  @pl.when(q < 3)
  def _():
    pltpu.sync_copy(outsh.at[pl.ds(hg * N + doff + DR, DQ - DR)],
                    out_hbm.at[pl.ds(hN + doff + DR, DQ - DR)])


def _s1(elT, erT, h1f, src, dst):
  return pl.kernel(
      _s1_body,
      out_type=jax.ShapeDtypeStruct((H1 * N, O1), f32),
      mesh=_MESH,
      compiler_params=_SC_PARAMS,
      scratch_types=[
          pltpu.VMEM((N,), f32),
          pltpu.VMEM((N,), f32),
          pltpu.VMEM((N,), f32),
          pltpu.VMEM((N,), f32),
          pltpu.VMEM((IB1,), i32),
          pltpu.VMEM((IB1,), i32),
          pltpu.VMEM((B1,), i32),
          pltpu.VMEM((B1,), i32),
          pltpu.VMEM((B1,), i32),
          pltpu.VMEM((B1,), i32),
          pltpu.VMEM((B1,), f32),
          pltpu.VMEM((B1, O1), f32),
          pltpu.VMEM((B1, O1), f32),
          pltpu.VMEM((ZB1R, O1), f32),
          pltpu.VMEM_SHARED((NS * N,), f32),
          pltpu.VMEM_SHARED((4 * N, O1), f32),
          pltpu.SemaphoreType.DMA,
          pltpu.SemaphoreType.DMA,
          pltpu.SemaphoreType.DMA,
          pltpu.SemaphoreType.DMA,
      ],
  )(elT, erT, h1f, src, dst)


def _s2_body(el2, er2, h2f, srcE, dstE, out_hbm,
             el_v, er_v, srs_v, tmp_v, srcbat, dstbat,
             iga, igb, isa, isb, alb, hra, hrb, zb,
             comb, outsh, semga, semgb, semsa, semsb):
  c = lax.axis_index("c")
  s = lax.axis_index("s")

  pltpu.sync_copy(el2, el_v)
  pltpu.sync_copy(er2, er_v)
  _zero_ref(srs_v, N // L)

  def p1(b, _):
    base = pl.multiple_of(s * ES2A + b * IB2, 8)
    pltpu.sync_copy(srcE.at[pl.ds(base, IB2)], srcbat)
    pltpu.sync_copy(dstE.at[pl.ds(base, IB2)], dstbat)

    def step(o):
      s16 = srcbat[pl.ds(o, L)]
      d16 = dstbat[pl.ds(o, L)]
      e = plsc.load_gather(el_v, [s16]) + plsc.load_gather(er_v, [d16])
      plsc.addupdate_scatter(srs_v, [d16], jnp.exp(e))

    _vec_loop(IB2 // L, step)
    return 0

  lax.fori_loop(0, NBT2A, p1, 0)

  sN = pl.multiple_of(s * N, 8)
  pltpu.sync_copy(srs_v, comb.at[pl.ds(sN, N)])
  plsc.subcore_barrier()
  pltpu.sync_copy(comb.at[pl.ds(0, N)], srs_v)
  for r in range(1, NS):
    gr = pl.multiple_of(r * N, 8)
    pltpu.sync_copy(comb.at[pl.ds(gr, N)], tmp_v)

    def acc(o):
      srs_v[pl.ds(o, L)] = srs_v[pl.ds(o, L)] + tmp_v[pl.ds(o, L)]

    _vec_loop(N // L, acc)

  def rcp(o):
    srs_v[pl.ds(o, L)] = 1.0 / (srs_v[pl.ds(o, L)] + 1e-9)

  _vec_loop(N // L, rcp)

  zero = jnp.zeros((L,), f32)

  def zrow(i, _):
    for k in range(CP // L):
      zb[i, pl.ds(k * L, L)] = zero
    return 0

  lax.fori_loop(0, ZR2, zrow, 0)
  zoff = s * ZC2
  pltpu.sync_copy(zb, outsh.at[pl.ds(zoff, ZR2)])

  @pl.when(s < NS - 1)
  def _():
    pltpu.sync_copy(zb.at[pl.ds(0, ZC2 - ZR2)],
                    outsh.at[pl.ds(zoff + ZR2, ZC2 - ZR2)])

  plsc.subcore_barrier()

  gw = c * NS + s

  def issue_gather(j, ig, hr, sg):
    def mk(o):
      ig[pl.ds(o, L)] = srcbat[pl.ds(j * B2 + o, L)]

    _vec_loop(B2 // L, mk)
    pltpu.async_copy(h2f.at[ig], hr, sg)

  def do_block(j, ig, hr, sg, isd, ss):
    def alpha(o):
      s16 = srcbat[pl.ds(j * B2 + o, L)]
      d16 = dstbat[pl.ds(j * B2 + o, L)]
      e = plsc.load_gather(el_v, [s16]) + plsc.load_gather(er_v, [d16])
      alb[pl.ds(o, L)] = jnp.exp(e) * plsc.load_gather(srs_v, [d16])

    _vec_loop(B2 // L, alpha)
    pltpu.make_async_copy(h2f.at[ig], hr, sg).wait()

    def mul(o):
      for k in range(L):
        bc = plsc.load_gather(alb, [jnp.full((L,), o + k, dtype=i32)])
        for kk in range(CP // L):
          sl = pl.ds(kk * L, L)
          hr[o + k, sl] = hr[o + k, sl] * bc

    _vec_loop(B2 // L, mul)

    def mkdst(o):
      isd[pl.ds(o, L)] = dstbat[pl.ds(j * B2 + o, L)]

    _vec_loop(B2 // L, mkdst)
    pltpu.async_copy(hr, outsh.at[isd], ss, add=True)

  def p3(b, _):
    base = pl.multiple_of(gw * ES2B + b * IB2, 8)
    pltpu.sync_copy(srcE.at[pl.ds(base, IB2)], srcbat)
    pltpu.sync_copy(dstE.at[pl.ds(base, IB2)], dstbat)
    issue_gather(0, iga, hra, semga)

    def pair(i, _):
      j0 = 2 * i

      @pl.when(i > 0)
      def _():
        pltpu.make_async_copy(hrb, outsh.at[isb], semsb).wait()

      issue_gather(j0 + 1, igb, hrb, semgb)
      do_block(j0, iga, hra, semga, isa, semsa)
      do_block(j0 + 1, igb, hrb, semgb, isb, semsb)
      pltpu.make_async_copy(hra, outsh.at[isa], semsa).wait()
      issue_gather(j0 + 2, iga, hra, semga)
      return 0

    lax.fori_loop(0, BPB2 // 2, pair, 0)
    pltpu.make_async_copy(hrb, outsh.at[isb], semsb).wait()
    do_block(BPB2 - 1, iga, hra, semga, isa, semsa)
    pltpu.make_async_copy(hra, outsh.at[isa], semsa).wait()
    return 0

  lax.fori_loop(0, NBT2B, p3, 0)
  plsc.subcore_barrier()

  pltpu.sync_copy(outsh.at[pl.ds(zoff, ZR2)],
                  out_hbm.at[pl.ds(c * N + zoff, ZR2)])

  @pl.when(s < NS - 1)
  def _():
    pltpu.sync_copy(outsh.at[pl.ds(zoff + ZR2, ZC2 - ZR2)],
                    out_hbm.at[pl.ds(c * N + zoff + ZR2, ZC2 - ZR2)])


def _s2(el2, er2, h2f, src, dst):
  return pl.kernel(
      _s2_body,
      out_type=jax.ShapeDtypeStruct((NC * N, CP), f32),
      mesh=_MESH,
      compiler_params=_SC_PARAMS,
      scratch_types=[
          pltpu.VMEM((N,), f32),
          pltpu.VMEM((N,), f32),
          pltpu.VMEM((N,), f32),
          pltpu.VMEM((N,), f32),
          pltpu.VMEM((IB2,), i32),
          pltpu.VMEM((IB2,), i32),
          pltpu.VMEM((B2,), i32),
          pltpu.VMEM((B2,), i32),
          pltpu.VMEM((B2,), i32),
          pltpu.VMEM((B2,), i32),
          pltpu.VMEM((B2,), f32),
          pltpu.VMEM((B2, CP), f32),
          pltpu.VMEM((B2, CP), f32),
          pltpu.VMEM((ZR2, CP), f32),
          pltpu.VMEM_SHARED((NS * N,), f32),
          pltpu.VMEM_SHARED((N, CP), f32),
          pltpu.SemaphoreType.DMA,
          pltpu.SemaphoreType.DMA,
          pltpu.SemaphoreType.DMA,
          pltpu.SemaphoreType.DMA,
      ],
  )(el2, er2, h2f, src, dst)


def kernel(feat, W1, attn_l1, attn_r1, bias1, W2, attn_l2, attn_r2, bias2,
           edge_index):
  src = edge_index[0]
  dst = edge_index[1]

  eye8 = jnp.eye(H1, dtype=f32)
  Gl1 = (eye8[:, None, :] * attn_l1[:, :, None]).reshape(D, H1)
  Gr1 = (eye8[:, None, :] * attn_r1[:, :, None]).reshape(D, H1)
  G1 = jnp.concatenate([Gl1, Gr1], axis=1)

  h1, eler1 = _t1(feat, W1, G1)
  elT1 = eler1[:, :H1].T.reshape(-1)
  erT1 = eler1[:, H1:].T.reshape(-1)
  h1f = h1.reshape(N, H1, O1).transpose(1, 0, 2).reshape(H1 * N, O1)

  agg1f = _s1(elT1, erT1, h1f, src, dst)
  agg1 = agg1f.reshape(H1, N, O1).transpose(1, 0, 2).reshape(N, D)

  W2p = jnp.pad(W2, ((0, 0), (0, CP - C)))
  G2 = jnp.zeros((CP, 8), f32)
  G2 = G2.at[:C, 0].set(attn_l2[0]).at[:C, 1].set(attn_r2[0])
  h2p, eler2 = _t2(agg1, bias1.reshape(1, D), W2p, G2)

  out2 = _s2(eler2[:, 0], eler2[:, 1], h2p, src, dst)

  b2p = jnp.pad(bias2, (0, CP - C)).reshape(1, CP)
  out48 = _t3(out2[:N], out2[N:], b2p)
  return out48[:, :C]


# speedup vs baseline: 38.5840x; 1.0443x over previous
import functools

import jax
import jax.numpy as jnp
from jax import lax
from jax.experimental import pallas as pl
from jax.experimental.pallas import tpu as pltpu
from jax.experimental.pallas import tpu_sc as plsc

N = 10000
E = 320000
D = 128
H1 = 8
O1 = 16
C = 40
CP = 48

NC = 2
NS = 16
L = 16

B1 = 128
ET1 = E // 4
NB1 = ET1 // B1
IB1 = 3200
NBT1 = ET1 // IB1
BPB1 = IB1 // B1
NQ = N // 4
ZC1 = 2504
ZR1 = 2440
ZB1R = 488
DQ = 2504
DR = 2488

B2 = 80
ES2A = E // NS
ES2B = E // (NC * NS)
IB2 = 2000
NBT2A = ES2A // IB2
NBT2B = ES2B // IB2
BPB2 = IB2 // B2
ZC2 = 632
ZR2 = 520

f32 = jnp.float32
i32 = jnp.int32

_MESH = plsc.VectorSubcoreMesh(
    core_axis_name="c", subcore_axis_name="s", num_cores=NC, num_subcores=NS
)
_SC_PARAMS = pltpu.CompilerParams(
    needs_layout_passes=False, use_tc_tiling_on_sc=False
)


def _zero_ref(ref, nrows):
  zero = jnp.zeros((L,), f32)

  def body(i, _):
    ref[pl.ds(i * L, L)] = zero
    return 0

  lax.fori_loop(0, nrows, body, 0)


def _vec_loop(n, body):
  def wrap(i, _):
    body(i * L)
    return 0

  lax.fori_loop(0, n, wrap, 0)


def _t1_body(feat_ref, w1_ref, g1_ref, h_ref, eler_ref):
  h = jnp.dot(feat_ref[...], w1_ref[...], preferred_element_type=f32)
  h_ref[...] = h
  eler_ref[...] = jnp.dot(h, g1_ref[...], preferred_element_type=f32)


def _t1(feat, W1, G1):
  nb = 2000
  return pl.pallas_call(
      _t1_body,
      grid=(N // nb,),
      in_specs=[
          pl.BlockSpec((nb, D), lambda i: (i, 0)),
          pl.BlockSpec((D, D), lambda i: (0, 0)),
          pl.BlockSpec((D, 2 * H1), lambda i: (0, 0)),
      ],
      out_specs=[
          pl.BlockSpec((nb, D), lambda i: (i, 0)),
          pl.BlockSpec((nb, 2 * H1), lambda i: (i, 0)),
      ],
      out_shape=[
          jax.ShapeDtypeStruct((N, D), f32),
          jax.ShapeDtypeStruct((N, 2 * H1), f32),
      ],
  )(feat, W1, G1)


def _t2_body(agg_ref, b1_ref, w2_ref, g2_ref, h2_ref, eler2_ref):
  x = agg_ref[...] + b1_ref[...]
  x = jnp.where(x > 0, x, jnp.exp(x) - 1.0)
  h2 = jnp.dot(x, w2_ref[...], preferred_element_type=f32)
  h2_ref[...] = h2
  eler2_ref[...] = jnp.dot(h2, g2_ref[...], preferred_element_type=f32)


def _t2(agg1, bias1, W2p, G2):
  nb = 2000
  return pl.pallas_call(
      _t2_body,
      grid=(N // nb,),
      in_specs=[
          pl.BlockSpec((nb, D), lambda i: (i, 0)),
          pl.BlockSpec((1, D), lambda i: (0, 0)),
          pl.BlockSpec((D, CP), lambda i: (0, 0)),
          pl.BlockSpec((CP, 8), lambda i: (0, 0)),
      ],
      out_specs=[
          pl.BlockSpec((nb, CP), lambda i: (i, 0)),
          pl.BlockSpec((nb, 8), lambda i: (i, 0)),
      ],
      out_shape=[
          jax.ShapeDtypeStruct((N, CP), f32),
          jax.ShapeDtypeStruct((N, 8), f32),
      ],
  )(agg1, bias1, W2p, G2)


def _t3_body(p0_ref, p1_ref, b2_ref, out_ref):
  x = p0_ref[...] + p1_ref[...] + b2_ref[...]
  out_ref[...] = jnp.where(x > 0, x, jnp.exp(x) - 1.0)


def _t3(p0, p1, b2p):
  nb = 2000
  return pl.pallas_call(
      _t3_body,
      grid=(N // nb,),
      in_specs=[
          pl.BlockSpec((nb, CP), lambda i: (i, 0)),
          pl.BlockSpec((nb, CP), lambda i: (i, 0)),
          pl.BlockSpec((1, CP), lambda i: (0, 0)),
      ],
      out_specs=pl.BlockSpec((nb, CP), lambda i: (i, 0)),
      out_shape=jax.ShapeDtypeStruct((N, CP), f32),
  )(p0, p1, b2p)


def _s1_body(elT, erT, h1f, srcE, dstE, out_hbm,
             el_v, er_v, srs_v, tmp_v, srcbat, dstbat,
             iga, igb, isa, isb, alb, hra, hrb, zb,
             comb, outsh, semga, semgb, semsa, semsb):
  c = lax.axis_index("c")
  s = lax.axis_index("s")
  hg = s // 4
  q = s % 4
  h = c * 4 + hg
  hN = pl.multiple_of(h * N, 8)

  pltpu.sync_copy(elT.at[pl.ds(hN, N)], el_v)
  pltpu.sync_copy(erT.at[pl.ds(hN, N)], er_v)
  _zero_ref(srs_v, N // L)

  def expv(o):
    el_v[pl.ds(o, L)] = jnp.exp(el_v[pl.ds(o, L)])
    er_v[pl.ds(o, L)] = jnp.exp(er_v[pl.ds(o, L)])

  _vec_loop(N // L, expv)

  def p1(b, _):
    base = pl.multiple_of(q * ET1 + b * IB1, 8)
    pltpu.sync_copy(srcE.at[pl.ds(base, IB1)], srcbat)
    pltpu.sync_copy(dstE.at[pl.ds(base, IB1)], dstbat)

    def step(o):
      s16 = srcbat[pl.ds(o, L)]
      d16 = dstbat[pl.ds(o, L)]
      e = plsc.load_gather(el_v, [s16]) * plsc.load_gather(er_v, [d16])
      plsc.addupdate_scatter(srs_v, [d16], e)

    _vec_loop(IB1 // L, step)
    return 0

  lax.fori_loop(0, NBT1, p1, 0)

  sN = pl.multiple_of(s * N, 8)
  pltpu.sync_copy(srs_v, comb.at[pl.ds(sN, N)])
  plsc.subcore_barrier()
  g0 = pl.multiple_of(4 * hg * N, 8)
  pltpu.sync_copy(comb.at[pl.ds(g0, N)], srs_v)
  for r in range(1, 4):
    gr = pl.multiple_of((4 * hg + r) * N, 8)
    pltpu.sync_copy(comb.at[pl.ds(gr, N)], tmp_v)

    def acc(o):
      srs_v[pl.ds(o, L)] = srs_v[pl.ds(o, L)] + tmp_v[pl.ds(o, L)]

    _vec_loop(N // L, acc)

  def rcp(o):
    er_v[pl.ds(o, L)] = er_v[pl.ds(o, L)] / (srs_v[pl.ds(o, L)] + 1e-9)

  _vec_loop(N // L, rcp)

  zero = jnp.zeros((L,), f32)

  def zrow(i, _):
    zb[i] = zero
    return 0

  lax.fori_loop(0, ZB1R, zrow, 0)
  zoff = s * ZC1

  def zcp(i, _):
    pltpu.sync_copy(zb, outsh.at[pl.ds(zoff + i * ZB1R, ZB1R)])
    return 0

  lax.fori_loop(0, ZR1 // ZB1R, zcp, 0)

  @pl.when(s < NS - 1)
  def _():
    pltpu.sync_copy(zb.at[pl.ds(0, ZC1 - ZR1)],
                    outsh.at[pl.ds(zoff + ZR1, ZC1 - ZR1)])

  plsc.subcore_barrier()

  def issue_gather(j, ig, hr, sg):
    def mk(o):
      ig[pl.ds(o, L)] = srcbat[pl.ds(j * B1 + o, L)] + hN

    _vec_loop(B1 // L, mk)
    pltpu.async_copy(h1f.at[ig], hr, sg)

  def do_block(j, ig, hr, sg, isd, ss):
    def alpha(o):
      s16 = srcbat[pl.ds(j * B1 + o, L)]
      d16 = dstbat[pl.ds(j * B1 + o, L)]
      alb[pl.ds(o, L)] = (
          plsc.load_gather(el_v, [s16]) * plsc.load_gather(er_v, [d16]))

    _vec_loop(B1 // L, alpha)
    pltpu.make_async_copy(h1f.at[ig], hr, sg).wait()

    def mul(o):
      for k in range(L):
        bc = plsc.load_gather(alb, [jnp.full((L,), o + k, dtype=i32)])
        hr[o + k] = hr[o + k] * bc

    _vec_loop(B1 // L, mul)

    def mkdst(o):
      isd[pl.ds(o, L)] = dstbat[pl.ds(j * B1 + o, L)] + hg * N

    _vec_loop(B1 // L, mkdst)
    pltpu.async_copy(hr, outsh.at[isd], ss, add=True)

  def p3(b, _):
    base = pl.multiple_of(q * ET1 + b * IB1, 8)
    pltpu.sync_copy(srcE.at[pl.ds(base, IB1)], srcbat)
    pltpu.sync_copy(dstE.at[pl.ds(base, IB1)], dstbat)
    issue_gather(0, iga, hra, semga)

    def pair(i, _):
      j0 = 2 * i

      @pl.when(i > 0)
      def _():
        pltpu.make_async_copy(hrb, outsh.at[isb], semsb).wait()

      issue_gather(j0 + 1, igb, hrb, semgb)
      do_block(j0, iga, hra, semga, isa, semsa)
      do_block(j0 + 1, igb, hrb, semgb, isb, semsb)
      pltpu.make_async_copy(hra, outsh.at[isa], semsa).wait()
      issue_gather(j0 + 2, iga, hra, semga)
      return 0

    lax.fori_loop(0, BPB1 // 2, pair, 0)
    pltpu.make_async_copy(hrb, outsh.at[isb], semsb).wait()
    do_block(BPB1 - 1, iga, hra, semga, isa, semsa)
    pltpu.make_async_copy(hra, outsh.at[isa], semsa).wait()
    return 0

  lax.fori_loop(0, NBT1, p3, 0)
  plsc.subcore_barrier()

  doff = q * DQ
  pltpu.sync_copy(outsh.at[pl.ds(hg * N + doff, DR)],
                  out_hbm.at[pl.ds(hN + doff, DR)])

  @pl.when(q < 3)
  def _():
    pltpu.sync_copy(outsh.at[pl.ds(hg * N + doff + DR, DQ - DR)],
                    out_hbm.at[pl.ds(hN + doff + DR, DQ - DR)])


def _s1(elT, erT, h1f, src, dst):
  return pl.kernel(
      _s1_body,
      out_type=jax.ShapeDtypeStruct((H1 * N, O1), f32),
      mesh=_MESH,
      compiler_params=_SC_PARAMS,
      scratch_types=[
          pltpu.VMEM((N,), f32),
          pltpu.VMEM((N,), f32),
          pltpu.VMEM((N,), f32),
          pltpu.VMEM((N,), f32),
          pltpu.VMEM((IB1,), i32),
          pltpu.VMEM((IB1,), i32),
          pltpu.VMEM((B1,), i32),
          pltpu.VMEM((B1,), i32),
          pltpu.VMEM((B1,), i32),
          pltpu.VMEM((B1,), i32),
          pltpu.VMEM((B1,), f32),
          pltpu.VMEM((B1, O1), f32),
          pltpu.VMEM((B1, O1), f32),
          pltpu.VMEM((ZB1R, O1), f32),
          pltpu.VMEM_SHARED((NS * N,), f32),
          pltpu.VMEM_SHARED((4 * N, O1), f32),
          pltpu.SemaphoreType.DMA,
          pltpu.SemaphoreType.DMA,
          pltpu.SemaphoreType.DMA,
          pltpu.SemaphoreType.DMA,
      ],
  )(elT, erT, h1f, src, dst)


def _s2_body(el2, er2, h2f, srcE, dstE, out_hbm,
             el_v, er_v, srs_v, tmp_v, srcbat, dstbat,
             iga, igb, isa, isb, alb, hra, hrb, zb,
             comb, outsh, semga, semgb, semsa, semsb):
  c = lax.axis_index("c")
  s = lax.axis_index("s")

  pltpu.sync_copy(el2, el_v)
  pltpu.sync_copy(er2, er_v)
  _zero_ref(srs_v, N // L)

  def expv(o):
    el_v[pl.ds(o, L)] = jnp.exp(el_v[pl.ds(o, L)])
    er_v[pl.ds(o, L)] = jnp.exp(er_v[pl.ds(o, L)])

  _vec_loop(N // L, expv)

  def p1(b, _):
    base = pl.multiple_of(s * ES2A + b * IB2, 8)
    pltpu.sync_copy(srcE.at[pl.ds(base, IB2)], srcbat)
    pltpu.sync_copy(dstE.at[pl.ds(base, IB2)], dstbat)

    def step(o):
      s16 = srcbat[pl.ds(o, L)]
      d16 = dstbat[pl.ds(o, L)]
      e = plsc.load_gather(el_v, [s16]) * plsc.load_gather(er_v, [d16])
      plsc.addupdate_scatter(srs_v, [d16], e)

    _vec_loop(IB2 // L, step)
    return 0

  lax.fori_loop(0, NBT2A, p1, 0)

  sN = pl.multiple_of(s * N, 8)
  pltpu.sync_copy(srs_v, comb.at[pl.ds(sN, N)])
  plsc.subcore_barrier()
  pltpu.sync_copy(comb.at[pl.ds(0, N)], srs_v)
  for r in range(1, NS):
    gr = pl.multiple_of(r * N, 8)
    pltpu.sync_copy(comb.at[pl.ds(gr, N)], tmp_v)

    def acc(o):
      srs_v[pl.ds(o, L)] = srs_v[pl.ds(o, L)] + tmp_v[pl.ds(o, L)]

    _vec_loop(N // L, acc)

  def rcp(o):
    er_v[pl.ds(o, L)] = er_v[pl.ds(o, L)] / (srs_v[pl.ds(o, L)] + 1e-9)

  _vec_loop(N // L, rcp)

  zero = jnp.zeros((L,), f32)

  def zrow(i, _):
    for k in range(CP // L):
      zb[i, pl.ds(k * L, L)] = zero
    return 0

  lax.fori_loop(0, ZR2, zrow, 0)
  zoff = s * ZC2
  pltpu.sync_copy(zb, outsh.at[pl.ds(zoff, ZR2)])

  @pl.when(s < NS - 1)
  def _():
    pltpu.sync_copy(zb.at[pl.ds(0, ZC2 - ZR2)],
                    outsh.at[pl.ds(zoff + ZR2, ZC2 - ZR2)])

  plsc.subcore_barrier()

  gw = c * NS + s

  def issue_gather(j, ig, hr, sg):
    def mk(o):
      ig[pl.ds(o, L)] = srcbat[pl.ds(j * B2 + o, L)]

    _vec_loop(B2 // L, mk)
    pltpu.async_copy(h2f.at[ig], hr, sg)

  def do_block(j, ig, hr, sg, isd, ss):
    def alpha(o):
      s16 = srcbat[pl.ds(j * B2 + o, L)]
      d16 = dstbat[pl.ds(j * B2 + o, L)]
      alb[pl.ds(o, L)] = (
          plsc.load_gather(el_v, [s16]) * plsc.load_gather(er_v, [d16]))

    _vec_loop(B2 // L, alpha)
    pltpu.make_async_copy(h2f.at[ig], hr, sg).wait()

    def mul(o):
      for k in range(L):
        bc = plsc.load_gather(alb, [jnp.full((L,), o + k, dtype=i32)])
        for kk in range(CP // L):
          sl = pl.ds(kk * L, L)
          hr[o + k, sl] = hr[o + k, sl] * bc

    _vec_loop(B2 // L, mul)

    def mkdst(o):
      isd[pl.ds(o, L)] = dstbat[pl.ds(j * B2 + o, L)]

    _vec_loop(B2 // L, mkdst)
    pltpu.async_copy(hr, outsh.at[isd], ss, add=True)

  def p3(b, _):
    base = pl.multiple_of(gw * ES2B + b * IB2, 8)
    pltpu.sync_copy(srcE.at[pl.ds(base, IB2)], srcbat)
    pltpu.sync_copy(dstE.at[pl.ds(base, IB2)], dstbat)
    issue_gather(0, iga, hra, semga)

    def pair(i, _):
      j0 = 2 * i

      @pl.when(i > 0)
      def _():
        pltpu.make_async_copy(hrb, outsh.at[isb], semsb).wait()

      issue_gather(j0 + 1, igb, hrb, semgb)
      do_block(j0, iga, hra, semga, isa, semsa)
      do_block(j0 + 1, igb, hrb, semgb, isb, semsb)
      pltpu.make_async_copy(hra, outsh.at[isa], semsa).wait()
      issue_gather(j0 + 2, iga, hra, semga)
      return 0

    lax.fori_loop(0, BPB2 // 2, pair, 0)
    pltpu.make_async_copy(hrb, outsh.at[isb], semsb).wait()
    do_block(BPB2 - 1, iga, hra, semga, isa, semsa)
    pltpu.make_async_copy(hra, outsh.at[isa], semsa).wait()
    return 0

  lax.fori_loop(0, NBT2B, p3, 0)
  plsc.subcore_barrier()

  pltpu.sync_copy(outsh.at[pl.ds(zoff, ZR2)],
                  out_hbm.at[pl.ds(c * N + zoff, ZR2)])

  @pl.when(s < NS - 1)
  def _():
    pltpu.sync_copy(outsh.at[pl.ds(zoff + ZR2, ZC2 - ZR2)],
                    out_hbm.at[pl.ds(c * N + zoff + ZR2, ZC2 - ZR2)])


def _s2(el2, er2, h2f, src, dst):
  return pl.kernel(
      _s2_body,
      out_type=jax.ShapeDtypeStruct((NC * N, CP), f32),
      mesh=_MESH,
      compiler_params=_SC_PARAMS,
      scratch_types=[
          pltpu.VMEM((N,), f32),
          pltpu.VMEM((N,), f32),
          pltpu.VMEM((N,), f32),
          pltpu.VMEM((N,), f32),
          pltpu.VMEM((IB2,), i32),
          pltpu.VMEM((IB2,), i32),
          pltpu.VMEM((B2,), i32),
          pltpu.VMEM((B2,), i32),
          pltpu.VMEM((B2,), i32),
          pltpu.VMEM((B2,), i32),
          pltpu.VMEM((B2,), f32),
          pltpu.VMEM((B2, CP), f32),
          pltpu.VMEM((B2, CP), f32),
          pltpu.VMEM((ZR2, CP), f32),
          pltpu.VMEM_SHARED((NS * N,), f32),
          pltpu.VMEM_SHARED((N, CP), f32),
          pltpu.SemaphoreType.DMA,
          pltpu.SemaphoreType.DMA,
          pltpu.SemaphoreType.DMA,
          pltpu.SemaphoreType.DMA,
      ],
  )(el2, er2, h2f, src, dst)


def kernel(feat, W1, attn_l1, attn_r1, bias1, W2, attn_l2, attn_r2, bias2,
           edge_index):
  src = edge_index[0]
  dst = edge_index[1]

  eye8 = jnp.eye(H1, dtype=f32)
  Gl1 = (eye8[:, None, :] * attn_l1[:, :, None]).reshape(D, H1)
  Gr1 = (eye8[:, None, :] * attn_r1[:, :, None]).reshape(D, H1)
  G1 = jnp.concatenate([Gl1, Gr1], axis=1)

  h1, eler1 = _t1(feat, W1, G1)
  elT1 = eler1[:, :H1].T.reshape(-1)
  erT1 = eler1[:, H1:].T.reshape(-1)
  h1f = h1.reshape(N, H1, O1).transpose(1, 0, 2).reshape(H1 * N, O1)

  agg1f = _s1(elT1, erT1, h1f, src, dst)
  agg1 = agg1f.reshape(H1, N, O1).transpose(1, 0, 2).reshape(N, D)

  W2p = jnp.pad(W2, ((0, 0), (0, CP - C)))
  G2 = jnp.zeros((CP, 8), f32)
  G2 = G2.at[:C, 0].set(attn_l2[0]).at[:C, 1].set(attn_r2[0])
  h2p, eler2 = _t2(agg1, bias1.reshape(1, D), W2p, G2)

  out2 = _s2(eler2[:, 0], eler2[:, 1], h2p, src, dst)

  b2p = jnp.pad(bias2, (0, CP - C)).reshape(1, CP)
  out48 = _t3(out2[:N], out2[N:], b2p)
  return out48[:, :C]


# speedup vs baseline: 44.2631x; 1.1472x over previous
import functools

import jax
import jax.numpy as jnp
from jax import lax
from jax.experimental import pallas as pl
from jax.experimental.pallas import tpu as pltpu
from jax.experimental.pallas import tpu_sc as plsc

N = 10000
E = 320000
D = 128
H1 = 8
O1 = 16
C = 40
CP = 48

NC = 2
NS = 16
L = 16

B1 = 128
ET1 = E // 4
NB1 = ET1 // B1
IB1 = 3200
NBT1 = ET1 // IB1
BPB1 = IB1 // B1
NQ = N // 4
ZC1 = 2504
ZR1 = 2440
ZB1R = 488
DQ = 2504
DR = 2488

B2 = 80
ES2A = E // NS
ES2B = E // (NC * NS)
IB2 = 2000
NBT2A = ES2A // IB2
NBT2B = ES2B // IB2
BPB2 = IB2 // B2
ZC2 = 632
ZR2 = 520

f32 = jnp.float32
i32 = jnp.int32

_MESH = plsc.VectorSubcoreMesh(
    core_axis_name="c", subcore_axis_name="s", num_cores=NC, num_subcores=NS
)
_SC_PARAMS = pltpu.CompilerParams(
    needs_layout_passes=False, use_tc_tiling_on_sc=False
)


def _zero_ref(ref, nrows):
  zero = jnp.zeros((L,), f32)

  def body(i, _):
    ref[pl.ds(i * L, L)] = zero
    return 0

  lax.fori_loop(0, nrows, body, 0)


def _vec_loop(n, body):
  def wrap(i, _):
    body(i * L)
    return 0

  lax.fori_loop(0, n, wrap, 0)


def _t1_body(feat_ref, w1_ref, g1_ref, h_ref, eler_ref):
  h = jnp.dot(feat_ref[...], w1_ref[...], preferred_element_type=f32)
  h_ref[...] = h
  eler_ref[...] = jnp.dot(h, g1_ref[...], preferred_element_type=f32)


def _t1(feat, W1, G1):
  nb = 2000
  return pl.pallas_call(
      _t1_body,
      grid=(N // nb,),
      in_specs=[
          pl.BlockSpec((nb, D), lambda i: (i, 0)),
          pl.BlockSpec((D, D), lambda i: (0, 0)),
          pl.BlockSpec((D, 2 * H1), lambda i: (0, 0)),
      ],
      out_specs=[
          pl.BlockSpec((nb, D), lambda i: (i, 0)),
          pl.BlockSpec((nb, 2 * H1), lambda i: (i, 0)),
      ],
      out_shape=[
          jax.ShapeDtypeStruct((N, D), f32),
          jax.ShapeDtypeStruct((N, 2 * H1), f32),
      ],
  )(feat, W1, G1)


def _t2_body(agg_ref, den_ref, qe_ref, b1_ref, w2_ref, g2_ref, h2_ref,
             eler2_ref):
  dd = jnp.dot(den_ref[...], qe_ref[...], preferred_element_type=f32)
  x = agg_ref[...] / dd + b1_ref[...]
  x = jnp.where(x > 0, x, jnp.exp(x) - 1.0)
  h2 = jnp.dot(x, w2_ref[...], preferred_element_type=f32)
  h2_ref[...] = h2
  eler2_ref[...] = jnp.dot(h2, g2_ref[...], preferred_element_type=f32)


def _t2(agg1, den32, QE, bias1, W2p, G2):
  nb = 2000
  return pl.pallas_call(
      _t2_body,
      grid=(N // nb,),
      in_specs=[
          pl.BlockSpec((nb, D), lambda i: (i, 0)),
          pl.BlockSpec((nb, NC * NS), lambda i: (i, 0)),
          pl.BlockSpec((NC * NS, D), lambda i: (0, 0)),
          pl.BlockSpec((1, D), lambda i: (0, 0)),
          pl.BlockSpec((D, CP), lambda i: (0, 0)),
          pl.BlockSpec((CP, 8), lambda i: (0, 0)),
      ],
      out_specs=[
          pl.BlockSpec((nb, CP), lambda i: (i, 0)),
          pl.BlockSpec((nb, 8), lambda i: (i, 0)),
      ],
      out_shape=[
          jax.ShapeDtypeStruct((N, CP), f32),
          jax.ShapeDtypeStruct((N, 8), f32),
      ],
  )(agg1, den32, QE, bias1, W2p, G2)


def _t3_body(p0_ref, p1_ref, den_ref, b2_ref, out_ref):
  d = jnp.sum(den_ref[...], axis=1, keepdims=True)
  x = (p0_ref[...] + p1_ref[...]) / d + b2_ref[...]
  out_ref[...] = jnp.where(x > 0, x, jnp.exp(x) - 1.0)


def _t3(p0, p1, den32, b2p):
  nb = 2000
  return pl.pallas_call(
      _t3_body,
      grid=(N // nb,),
      in_specs=[
          pl.BlockSpec((nb, CP), lambda i: (i, 0)),
          pl.BlockSpec((nb, CP), lambda i: (i, 0)),
          pl.BlockSpec((nb, NC * NS), lambda i: (i, 0)),
          pl.BlockSpec((1, CP), lambda i: (0, 0)),
      ],
      out_specs=pl.BlockSpec((nb, CP), lambda i: (i, 0)),
      out_shape=jax.ShapeDtypeStruct((N, CP), f32),
  )(p0, p1, den32, b2p)


def _s1_body(elT, erT, h1f, srcE, dstE, out_hbm, outs_hbm,
             el_v, er_v, srs_v, srcbat, dstbat,
             iga, igb, isa, isb, alb, hra, hrb, zb,
             outsh, semga, semgb, semsa, semsb):
  c = lax.axis_index("c")
  s = lax.axis_index("s")
  hg = s // 4
  q = s % 4
  h = c * 4 + hg
  hN = pl.multiple_of(h * N, 8)

  pltpu.sync_copy(elT.at[pl.ds(hN, N)], el_v)
  pltpu.sync_copy(erT.at[pl.ds(hN, N)], er_v)
  _zero_ref(srs_v, N // L)

  def expv(o):
    el_v[pl.ds(o, L)] = jnp.exp(el_v[pl.ds(o, L)])
    er_v[pl.ds(o, L)] = jnp.exp(er_v[pl.ds(o, L)])

  _vec_loop(N // L, expv)

  zero = jnp.zeros((L,), f32)

  def zrow(i, _):
    zb[i] = zero
    return 0

  lax.fori_loop(0, ZB1R, zrow, 0)
  zoff = s * ZC1

  def zcp(i, _):
    pltpu.sync_copy(zb, outsh.at[pl.ds(zoff + i * ZB1R, ZB1R)])
    return 0

  lax.fori_loop(0, ZR1 // ZB1R, zcp, 0)

  @pl.when(s < NS - 1)
  def _():
    pltpu.sync_copy(zb.at[pl.ds(0, ZC1 - ZR1)],
                    outsh.at[pl.ds(zoff + ZR1, ZC1 - ZR1)])

  plsc.subcore_barrier()

  def issue_gather(j, ig, hr, sg):
    def mk(o):
      ig[pl.ds(o, L)] = srcbat[pl.ds(j * B1 + o, L)] + hN

    _vec_loop(B1 // L, mk)
    pltpu.async_copy(h1f.at[ig], hr, sg)

  def do_block(j, ig, hr, sg, isd, ss):
    def alpha(o):
      s16 = srcbat[pl.ds(j * B1 + o, L)]
      d16 = dstbat[pl.ds(j * B1 + o, L)]
      w = plsc.load_gather(el_v, [s16]) * plsc.load_gather(er_v, [d16])
      alb[pl.ds(o, L)] = w
      plsc.addupdate_scatter(srs_v, [d16], w)

    _vec_loop(B1 // L, alpha)
    pltpu.make_async_copy(h1f.at[ig], hr, sg).wait()

    def mul(o):
      for k in range(L):
        bc = plsc.load_gather(alb, [jnp.full((L,), o + k, dtype=i32)])
        hr[o + k] = hr[o + k] * bc

    _vec_loop(B1 // L, mul)

    def mkdst(o):
      isd[pl.ds(o, L)] = dstbat[pl.ds(j * B1 + o, L)] + hg * N

    _vec_loop(B1 // L, mkdst)
    pltpu.async_copy(hr, outsh.at[isd], ss, add=True)

  def p3(b, _):
    base = pl.multiple_of(q * ET1 + b * IB1, 8)
    pltpu.sync_copy(srcE.at[pl.ds(base, IB1)], srcbat)
    pltpu.sync_copy(dstE.at[pl.ds(base, IB1)], dstbat)
    issue_gather(0, iga, hra, semga)

    def pair(i, _):
      j0 = 2 * i

      @pl.when(i > 0)
      def _():
        pltpu.make_async_copy(hrb, outsh.at[isb], semsb).wait()

      issue_gather(j0 + 1, igb, hrb, semgb)
      do_block(j0, iga, hra, semga, isa, semsa)
      do_block(j0 + 1, igb, hrb, semgb, isb, semsb)
      pltpu.make_async_copy(hra, outsh.at[isa], semsa).wait()
      issue_gather(j0 + 2, iga, hra, semga)
      return 0

    lax.fori_loop(0, BPB1 // 2, pair, 0)
    pltpu.make_async_copy(hrb, outsh.at[isb], semsb).wait()
    do_block(BPB1 - 1, iga, hra, semga, isa, semsa)
    pltpu.make_async_copy(hra, outsh.at[isa], semsa).wait()
    return 0

  lax.fori_loop(0, NBT1, p3, 0)
  plsc.subcore_barrier()

  tN = pl.multiple_of((c * NS + s) * N, 8)
  pltpu.sync_copy(srs_v, outs_hbm.at[pl.ds(tN, N)])
  doff = q * DQ
  pltpu.sync_copy(outsh.at[pl.ds(hg * N + doff, DR)],
                  out_hbm.at[pl.ds(hN + doff, DR)])

  @pl.when(q < 3)
  def _():
    pltpu.sync_copy(outsh.at[pl.ds(hg * N + doff + DR, DQ - DR)],
                    out_hbm.at[pl.ds(hN + doff + DR, DQ - DR)])


def _s1(elT, erT, h1f, src, dst):
  return pl.kernel(
      _s1_body,
      out_type=[
          jax.ShapeDtypeStruct((H1 * N, O1), f32),
          jax.ShapeDtypeStruct((NC * NS * N,), f32),
      ],
      mesh=_MESH,
      compiler_params=_SC_PARAMS,
      scratch_types=[
          pltpu.VMEM((N,), f32),
          pltpu.VMEM((N,), f32),
          pltpu.VMEM((N,), f32),
          pltpu.VMEM((IB1,), i32),
          pltpu.VMEM((IB1,), i32),
          pltpu.VMEM((B1,), i32),
          pltpu.VMEM((B1,), i32),
          pltpu.VMEM((B1,), i32),
          pltpu.VMEM((B1,), i32),
          pltpu.VMEM((B1,), f32),
          pltpu.VMEM((B1, O1), f32),
          pltpu.VMEM((B1, O1), f32),
          pltpu.VMEM((ZB1R, O1), f32),
          pltpu.VMEM_SHARED((4 * N, O1), f32),
          pltpu.SemaphoreType.DMA,
          pltpu.SemaphoreType.DMA,
          pltpu.SemaphoreType.DMA,
          pltpu.SemaphoreType.DMA,
      ],
  )(elT, erT, h1f, src, dst)


def _s2_body(el2, er2, h2f, srcE, dstE, out_hbm, outs_hbm,
             el_v, er_v, srs_v, srcbat, dstbat,
             iga, igb, isa, isb, alb, hra, hrb, zb,
             outsh, semga, semgb, semsa, semsb):
  c = lax.axis_index("c")
  s = lax.axis_index("s")

  pltpu.sync_copy(el2, el_v)
  pltpu.sync_copy(er2, er_v)
  _zero_ref(srs_v, N // L)

  def expv(o):
    el_v[pl.ds(o, L)] = jnp.exp(el_v[pl.ds(o, L)])
    er_v[pl.ds(o, L)] = jnp.exp(er_v[pl.ds(o, L)])

  _vec_loop(N // L, expv)

  zero = jnp.zeros((L,), f32)

  def zrow(i, _):
    for k in range(CP // L):
      zb[i, pl.ds(k * L, L)] = zero
    return 0

  lax.fori_loop(0, ZR2, zrow, 0)
  zoff = s * ZC2
  pltpu.sync_copy(zb, outsh.at[pl.ds(zoff, ZR2)])

  @pl.when(s < NS - 1)
  def _():
    pltpu.sync_copy(zb.at[pl.ds(0, ZC2 - ZR2)],
                    outsh.at[pl.ds(zoff + ZR2, ZC2 - ZR2)])

  plsc.subcore_barrier()

  gw = c * NS + s

  def issue_gather(j, ig, hr, sg):
    def mk(o):
      ig[pl.ds(o, L)] = srcbat[pl.ds(j * B2 + o, L)]

    _vec_loop(B2 // L, mk)
    pltpu.async_copy(h2f.at[ig], hr, sg)

  def do_block(j, ig, hr, sg, isd, ss):
    def alpha(o):
      s16 = srcbat[pl.ds(j * B2 + o, L)]
      d16 = dstbat[pl.ds(j * B2 + o, L)]
      w = plsc.load_gather(el_v, [s16]) * plsc.load_gather(er_v, [d16])
      alb[pl.ds(o, L)] = w
      plsc.addupdate_scatter(srs_v, [d16], w)

    _vec_loop(B2 // L, alpha)
    pltpu.make_async_copy(h2f.at[ig], hr, sg).wait()

    def mul(o):
      for k in range(L):
        bc = plsc.load_gather(alb, [jnp.full((L,), o + k, dtype=i32)])
        for kk in range(CP // L):
          sl = pl.ds(kk * L, L)
          hr[o + k, sl] = hr[o + k, sl] * bc

    _vec_loop(B2 // L, mul)

    def mkdst(o):
      isd[pl.ds(o, L)] = dstbat[pl.ds(j * B2 + o, L)]

    _vec_loop(B2 // L, mkdst)
    pltpu.async_copy(hr, outsh.at[isd], ss, add=True)

  def p3(b, _):
    base = pl.multiple_of(gw * ES2B + b * IB2, 8)
    pltpu.sync_copy(srcE.at[pl.ds(base, IB2)], srcbat)
    pltpu.sync_copy(dstE.at[pl.ds(base, IB2)], dstbat)
    issue_gather(0, iga, hra, semga)

    def pair(i, _):
      j0 = 2 * i

      @pl.when(i > 0)
      def _():
        pltpu.make_async_copy(hrb, outsh.at[isb], semsb).wait()

      issue_gather(j0 + 1, igb, hrb, semgb)
      do_block(j0, iga, hra, semga, isa, semsa)
      do_block(j0 + 1, igb, hrb, semgb, isb, semsb)
      pltpu.make_async_copy(hra, outsh.at[isa], semsa).wait()
      issue_gather(j0 + 2, iga, hra, semga)
      return 0

    lax.fori_loop(0, BPB2 // 2, pair, 0)
    pltpu.make_async_copy(hrb, outsh.at[isb], semsb).wait()
    do_block(BPB2 - 1, iga, hra, semga, isa, semsa)
    pltpu.make_async_copy(hra, outsh.at[isa], semsa).wait()
    return 0

  lax.fori_loop(0, NBT2B, p3, 0)
  plsc.subcore_barrier()

  tN = pl.multiple_of((c * NS + s) * N, 8)
  pltpu.sync_copy(srs_v, outs_hbm.at[pl.ds(tN, N)])
  pltpu.sync_copy(outsh.at[pl.ds(zoff, ZR2)],
                  out_hbm.at[pl.ds(c * N + zoff, ZR2)])

  @pl.when(s < NS - 1)
  def _():
    pltpu.sync_copy(outsh.at[pl.ds(zoff + ZR2, ZC2 - ZR2)],
                    out_hbm.at[pl.ds(c * N + zoff + ZR2, ZC2 - ZR2)])


def _s2(el2, er2, h2f, src, dst):
  return pl.kernel(
      _s2_body,
      out_type=[
          jax.ShapeDtypeStruct((NC * N, CP), f32),
          jax.ShapeDtypeStruct((NC * NS * N,), f32),
      ],
      mesh=_MESH,
      compiler_params=_SC_PARAMS,
      scratch_types=[
          pltpu.VMEM((N,), f32),
          pltpu.VMEM((N,), f32),
          pltpu.VMEM((N,), f32),
          pltpu.VMEM((IB2,), i32),
          pltpu.VMEM((IB2,), i32),
          pltpu.VMEM((B2,), i32),
          pltpu.VMEM((B2,), i32),
          pltpu.VMEM((B2,), i32),
          pltpu.VMEM((B2,), i32),
          pltpu.VMEM((B2,), f32),
          pltpu.VMEM((B2, CP), f32),
          pltpu.VMEM((B2, CP), f32),
          pltpu.VMEM((ZR2, CP), f32),
          pltpu.VMEM_SHARED((N, CP), f32),
          pltpu.SemaphoreType.DMA,
          pltpu.SemaphoreType.DMA,
          pltpu.SemaphoreType.DMA,
          pltpu.SemaphoreType.DMA,
      ],
  )(el2, er2, h2f, src, dst)


def kernel(feat, W1, attn_l1, attn_r1, bias1, W2, attn_l2, attn_r2, bias2,
           edge_index):
  src = edge_index[0]
  dst = edge_index[1]

  eye8 = jnp.eye(H1, dtype=f32)
  Gl1 = (eye8[:, None, :] * attn_l1[:, :, None]).reshape(D, H1)
  Gr1 = (eye8[:, None, :] * attn_r1[:, :, None]).reshape(D, H1)
  G1 = jnp.concatenate([Gl1, Gr1], axis=1)

  h1, eler1 = _t1(feat, W1, G1)
  elT1 = eler1[:, :H1].T.reshape(-1)
  erT1 = eler1[:, H1:].T.reshape(-1)
  h1f = h1.reshape(N, H1, O1).transpose(1, 0, 2).reshape(H1 * N, O1)

  agg1f, douts1 = _s1(elT1, erT1, h1f, src, dst)
  agg1 = agg1f.reshape(H1, N, O1).transpose(1, 0, 2).reshape(N, D)
  den32_1 = douts1.reshape(NC * NS, N).T

  tiles = jnp.arange(NC * NS, dtype=i32)
  tile_head = (tiles // NS) * 4 + (tiles % NS) // 4
  col_head = jnp.arange(D, dtype=i32) // O1
  QE = (tile_head[:, None] == col_head[None, :]).astype(f32)

  W2p = jnp.pad(W2, ((0, 0), (0, CP - C)))
  G2 = jnp.zeros((CP, 8), f32)
  G2 = G2.at[:C, 0].set(attn_l2[0]).at[:C, 1].set(attn_r2[0])
  h2p, eler2 = _t2(agg1, den32_1, QE, bias1.reshape(1, D), W2p, G2)

  out2, douts2 = _s2(eler2[:, 0], eler2[:, 1], h2p, src, dst)
  den32_2 = douts2.reshape(NC * NS, N).T

  b2p = jnp.pad(bias2, (0, CP - C)).reshape(1, CP)
  out48 = _t3(out2[:N], out2[N:], den32_2, b2p)
  return out48[:, :C]
